# baseline (speedup 1.0000x reference)
import functools

import jax
import jax.numpy as jnp
from jax import lax
from jax.experimental import pallas as pl
from jax.experimental.pallas import tpu as pltpu
from jax.experimental.pallas import tpu_sc as plsc

N_TOK = 4096
N_FEAT = 256
N_UNIT = 8192
TOPK_N = 10


BM = 256
BN = 2048


def _dist_body(x_ref, x2_ref, c_ref, c2_ref, o_ref):
    g = jax.lax.dot_general(
        x_ref[...], c_ref[...], (((1,), (1,)), ((), ())),
        preferred_element_type=jnp.float32,
    )
    d2 = jnp.maximum(x2_ref[...] - 2.0 * g + c2_ref[...], 0.0)
    o_ref[...] = jnp.sqrt(d2)


def _dist(x, x2, c, c2):
    grid = (N_TOK // BM, N_UNIT // BN)
    return pl.pallas_call(
        _dist_body,
        grid=grid,
        in_specs=[
            pl.BlockSpec((BM, N_FEAT), lambda i, j: (i, 0)),
            pl.BlockSpec((BM, 1), lambda i, j: (i, 0)),
            pl.BlockSpec((BN, N_FEAT), lambda i, j: (j, 0)),
            pl.BlockSpec((1, BN), lambda i, j: (0, j)),
        ],
        out_specs=pl.BlockSpec((BM, BN), lambda i, j: (i, j)),
        out_shape=jax.ShapeDtypeStruct((N_TOK, N_UNIT), jnp.float32),
    )(x, x2, c, c2)


NC = 2
NS = 16
NW = NC * NS
LANES = 16
RADIX = 256
NHIST = 4
XC = 32


def _make_sc_sort(n_tok, n_unit, n_feat, interpret=False):
    seg = n_unit // LANES
    pseg = seg + 1
    sub = seg // NHIST
    histn = RADIX * LANES
    nvec = n_unit // LANES
    vblk = seg // LANES
    rows_w = n_tok // NW

    mesh = plsc.VectorSubcoreMesh(
        core_axis_name="c", subcore_axis_name="s",
        num_cores=NC, num_subcores=NS)

    def body(d_hbm, c_hbm, isort_hbm, k_hbm, z_hbm, xc_hbm,
             d_buf, key0, ka, pa, kb, pb, pay_fin,
             h0, h1, h2, h3, lrank, r_tot, r_exc,
             k_buf, z_buf, nearest, xc_buf, sem,
             i_sem, k_sem, z_sem):
        hists = (h0, h1, h2, h3)
        cid = lax.axis_index("c")
        sid = lax.axis_index("s")
        wid = sid * NC + cid
        base_row = wid * rows_w

        lane = jnp.arange(LANES, dtype=jnp.int32)
        seg0n = lane * seg
        seg0p = lane * pseg
        ones = jnp.ones((LANES,), jnp.int32)
        zeros16 = jnp.zeros((LANES,), jnp.int32)
        lane0 = lane == 0
        zvals = jnp.where(lane < TOPK_N,
                          1.0 / (lane.astype(jnp.float32) + 1.0),
                          jnp.float32(0.0))
        fzeros = jnp.zeros((LANES,), jnp.float32)

        def do_pass(src_key, src_pay, dst_key, dst_pay, shift, first, last):
            @plsc.parallel_loop(0, RADIX, unroll=2)
            def zh(i):
                h0[pl.ds(i * LANES, LANES)] = zeros16
                h1[pl.ds(i * LANES, LANES)] = zeros16
                h2[pl.ds(i * LANES, LANES)] = zeros16
                h3[pl.ds(i * LANES, LANES)] = zeros16

            def count(t, carry):
                for u in range(NHIST):
                    idx = seg0p + (u * sub + t)
                    kv = plsc.load_gather(src_key, [idx])
                    dig = lax.shift_right_logical(kv, shift) & 0xFF
                    addr = dig * LANES + lane
                    old = plsc.load_gather(hists[u], [addr])
                    plsc.store_scatter(hists[u], [addr], old + ones)
                    lrank[pl.ds((t * NHIST + u) * LANES, LANES)] = old
                return carry
            lax.fori_loop(0, sub, count, 0, unroll=2)

            @plsc.parallel_loop(0, RADIX, unroll=4)
            def s_tot(g):
                v = ((h0[pl.ds(g * LANES, LANES)] + h1[pl.ds(g * LANES, LANES)])
                     + (h2[pl.ds(g * LANES, LANES)] + h3[pl.ds(g * LANES, LANES)]))
                incl = plsc.cumsum(v)
                plsc.store_scatter(r_tot, [zeros16 + g], incl,
                                   mask=lane == LANES - 1)

            def s_scan(b, acc):
                v = r_tot[pl.ds(b * LANES, LANES)]
                incl = plsc.cumsum(v)
                r_exc[pl.ds(b * LANES, LANES)] = acc + (incl - v)
                return acc + jnp.max(incl)
            lax.fori_loop(0, RADIX // LANES, s_scan, zeros16, unroll=2)

            @plsc.parallel_loop(0, RADIX, unroll=4)
            def s_wb(g):
                base = plsc.load_gather(r_exc, [zeros16 + g])
                c0 = h0[pl.ds(g * LANES, LANES)]
                c1 = h1[pl.ds(g * LANES, LANES)]
                c2 = h2[pl.ds(g * LANES, LANES)]
                c3 = h3[pl.ds(g * LANES, LANES)]
                v = (c0 + c1) + (c2 + c3)
                exl = plsc.cumsum(v) - v
                pre0 = base + exl
                pre1 = pre0 + c0
                pre2 = pre1 + c1
                h0[pl.ds(g * LANES, LANES)] = pre0
                h1[pl.ds(g * LANES, LANES)] = pre1
                h2[pl.ds(g * LANES, LANES)] = pre2
                h3[pl.ds(g * LANES, LANES)] = pre2 + c2

            @plsc.parallel_loop(0, sub, unroll=4)
            def perm(t):
                for u in range(NHIST):
                    off = u * sub + t
                    kv = plsc.load_gather(src_key, [seg0p + off])
                    pv = seg0n + off if first else plsc.load_gather(src_pay, [seg0p + off])
                    dig = lax.shift_right_logical(kv, shift) & 0xFF
                    pre = plsc.load_gather(hists[u], [dig * LANES + lane])
                    lr = lrank[pl.ds((t * NHIST + u) * LANES, LANES)]
                    dest = pre + lr
                    if last:
                        plsc.store_scatter(dst_pay, [dest], pv)
                    else:
                        dest_p = dest + lax.shift_right_logical(dest, 9)
                        plsc.store_scatter(dst_key, [dest_p], kv)
                        plsc.store_scatter(dst_pay, [dest_p], pv)

        def row_body(rr, carry):
            row = base_row + rr
            pltpu.sync_copy(d_hbm.at[row], d_buf)

            def repack(t, mm):
                mn, mx = mm
                kv = plsc.bitcast(d_buf[pl.ds(t * LANES, LANES)], jnp.int32)
                key0[pl.ds(t * LANES + lax.shift_right_logical(t, 5), LANES)] = kv
                return jnp.minimum(mn, kv), jnp.maximum(mx, kv)
            mn, mx = lax.fori_loop(0, nvec, repack,
                                   (jnp.full((LANES,), 0x7FFFFFFF, jnp.int32),
                                    zeros16), unroll=2)
            minb = jnp.min(mn)
            maxb = jnp.max(mx)
            need4 = lax.shift_right_logical(minb, 24) != lax.shift_right_logical(maxb, 24)

            do_pass(key0, None, kb, pb, 0, True, False)
            do_pass(kb, pb, ka, pa, 8, False, False)
            do_pass(ka, pa, kb, pb, 16, False, False)

            @pl.when(rr > 0)
            def _():
                pltpu.make_async_copy(pay_fin, isort_hbm.at[row - 1], i_sem).wait()
                pltpu.make_async_copy(k_buf, k_hbm.at[row - 1], k_sem).wait()
                pltpu.make_async_copy(z_buf, z_hbm.at[row - 1], z_sem).wait()

            @pl.when(need4)
            def _():
                do_pass(kb, pb, None, pay_fin, 24, False, True)

            @pl.when(jnp.logical_not(need4))
            def _():
                @plsc.parallel_loop(0, nvec, unroll=2)
                def depad(t):
                    pay_fin[pl.ds(t * LANES, LANES)] = \
                        pb[pl.ds(t * LANES + lax.shift_right_logical(t, 5), LANES)]

            pltpu.async_copy(pay_fin, isort_hbm.at[row], i_sem)

            @plsc.parallel_loop(0, nvec, unroll=2)
            def inv(t):
                u = pay_fin[pl.ds(t * LANES, LANES)]
                plsc.store_scatter(k_buf, [u], t * LANES + lane)

            @plsc.parallel_loop(0, nvec, unroll=2)
            def zfill(t):
                z_buf[pl.ds(t * LANES, LANES)] = fzeros

            top16 = pay_fin[pl.ds(0, LANES)]
            plsc.store_scatter(z_buf, [top16], zvals)

            pltpu.async_copy(k_buf, k_hbm.at[row], k_sem)
            pltpu.async_copy(z_buf, z_hbm.at[row], z_sem)

            plsc.store_scatter(nearest, [zeros16 + rr], top16, mask=lane0)
            return carry
        lax.fori_loop(0, rows_w, row_body, 0)

        last_row = base_row + rows_w - 1
        pltpu.make_async_copy(pay_fin, isort_hbm.at[last_row], i_sem).wait()
        pltpu.make_async_copy(k_buf, k_hbm.at[last_row], k_sem).wait()
        pltpu.make_async_copy(z_buf, z_hbm.at[last_row], z_sem).wait()

        def xg(t, carry):
            pltpu.async_copy(
                c_hbm.at[nearest.at[pl.ds(t * XC, XC)]], xc_buf, sem).wait()
            pltpu.sync_copy(xc_buf, xc_hbm.at[pl.ds(base_row + t * XC, XC)])
            return carry
        lax.fori_loop(0, rows_w // XC, xg, 0)

    return pl.kernel(
        body,
        out_type=[
            jax.ShapeDtypeStruct((n_tok, n_unit), jnp.int32),
            jax.ShapeDtypeStruct((n_tok, n_unit), jnp.int32),
            jax.ShapeDtypeStruct((n_tok, n_unit), jnp.float32),
            jax.ShapeDtypeStruct((n_tok, n_feat), jnp.float32),
        ],
        mesh=mesh,
        scratch_types=[
            pltpu.VMEM((n_unit,), jnp.float32),
            pltpu.VMEM((LANES * pseg,), jnp.int32),
            pltpu.VMEM((LANES * pseg,), jnp.int32),
            pltpu.VMEM((LANES * pseg,), jnp.int32),
            pltpu.VMEM((LANES * pseg,), jnp.int32),
            pltpu.VMEM((LANES * pseg,), jnp.int32),
            pltpu.VMEM((n_unit,), jnp.int32),
            pltpu.VMEM((histn,), jnp.int32),
            pltpu.VMEM((histn,), jnp.int32),
            pltpu.VMEM((histn,), jnp.int32),
            pltpu.VMEM((histn,), jnp.int32),
            pltpu.VMEM((n_unit,), jnp.int32),
            pltpu.VMEM((RADIX,), jnp.int32),
            pltpu.VMEM((RADIX,), jnp.int32),
            pltpu.VMEM((n_unit,), jnp.int32),
            pltpu.VMEM((n_unit,), jnp.float32),
            pltpu.VMEM((rows_w,), jnp.int32),
            pltpu.VMEM((XC, n_feat), jnp.float32),
            pltpu.SemaphoreType.DMA,
            pltpu.SemaphoreType.DMA,
            pltpu.SemaphoreType.DMA,
            pltpu.SemaphoreType.DMA,
        ],
        compiler_params=pltpu.CompilerParams(needs_layout_passes=False),
        interpret=interpret,
    )


def kernel(x, c):
    x2 = jnp.sum(x * x, axis=-1, keepdims=True)
    c2 = jnp.sum(c * c, axis=-1)[None, :]
    d = _dist(x, x2, c, c2)
    sc = _make_sc_sort(N_TOK, N_UNIT, N_FEAT)
    i_sort, k, z, x_c = sc(d, c)
    return (d, i_sort, k, z, x_c)


# speedup vs baseline: 1.4486x; 1.4486x over previous
import functools

import jax
import jax.numpy as jnp
from jax import lax
from jax.experimental import pallas as pl
from jax.experimental.pallas import tpu as pltpu
from jax.experimental.pallas import tpu_sc as plsc

N_TOK = 4096
N_FEAT = 256
N_UNIT = 8192
TOPK_N = 10


BM = 256
BN = 2048


def _dist_body(x_ref, x2_ref, c_ref, c2_ref, o_ref):
    g = jax.lax.dot_general(
        x_ref[...], c_ref[...], (((1,), (1,)), ((), ())),
        preferred_element_type=jnp.float32,
    )
    d2 = jnp.maximum(x2_ref[...] - 2.0 * g + c2_ref[...], 0.0)
    o_ref[...] = jnp.sqrt(d2)


def _dist(x, x2, c, c2):
    grid = (N_TOK // BM, N_UNIT // BN)
    return pl.pallas_call(
        _dist_body,
        grid=grid,
        in_specs=[
            pl.BlockSpec((BM, N_FEAT), lambda i, j: (i, 0)),
            pl.BlockSpec((BM, 1), lambda i, j: (i, 0)),
            pl.BlockSpec((BN, N_FEAT), lambda i, j: (j, 0)),
            pl.BlockSpec((1, BN), lambda i, j: (0, j)),
        ],
        out_specs=pl.BlockSpec((BM, BN), lambda i, j: (i, j)),
        out_shape=jax.ShapeDtypeStruct((N_TOK, N_UNIT), jnp.float32),
    )(x, x2, c, c2)


NC = 2
NS = 16
NW = NC * NS
LANES = 16
RADIX = 256
NHIST = 4
XC = 32


def _make_sc_sort(n_tok, n_unit, n_feat, interpret=False):
    seg = n_unit // LANES
    pseg = seg + 1
    sub = seg // NHIST
    histn = RADIX * LANES
    nvec = n_unit // LANES
    vblk = seg // LANES
    rows_w = n_tok // NW

    mesh = plsc.VectorSubcoreMesh(
        core_axis_name="c", subcore_axis_name="s",
        num_cores=NC, num_subcores=NS)

    def body(d_hbm, c_hbm, isort_hbm, k_hbm, z_hbm, xc_hbm,
             d_buf, key0, ka, pa, kb, pb, pay_fin,
             h0, h1, h2, h3, lrank, r_tot, r_exc,
             k_buf, z_buf, nearest, xc_buf, sem,
             i_sem, k_sem, z_sem):
        hists = (h0, h1, h2, h3)
        cid = lax.axis_index("c")
        sid = lax.axis_index("s")
        wid = sid * NC + cid
        base_row = wid * rows_w

        lane = jnp.arange(LANES, dtype=jnp.int32)
        seg0n = lane * seg
        seg0p = lane * pseg
        ones = jnp.ones((LANES,), jnp.int32)
        zeros16 = jnp.zeros((LANES,), jnp.int32)
        lane0 = lane == 0
        zvals = jnp.where(lane < TOPK_N,
                          1.0 / (lane.astype(jnp.float32) + 1.0),
                          jnp.float32(0.0))
        fzeros = jnp.zeros((LANES,), jnp.float32)

        def do_pass(src_key, src_pay, dst_key, dst_pay, shift, first, last):
            @plsc.parallel_loop(0, RADIX, unroll=2)
            def zh(i):
                h0[pl.ds(i * LANES, LANES)] = zeros16
                h1[pl.ds(i * LANES, LANES)] = zeros16
                h2[pl.ds(i * LANES, LANES)] = zeros16
                h3[pl.ds(i * LANES, LANES)] = zeros16

            def count(i, carry):
                t = i * 2
                kvs = []
                for u in range(NHIST):
                    for w in range(2):
                        idx = seg0p + (u * sub + t + w)
                        kv = plsc.load_gather(src_key, [idx])
                        dig = lax.shift_right_logical(kv, shift) & 0xFF
                        kvs.append(dig * LANES + lane)
                for u in range(NHIST):
                    a1 = kvs[2 * u]
                    a2 = kvs[2 * u + 1]
                    o1 = plsc.load_gather(hists[u], [a1])
                    o2 = plsc.load_gather(hists[u], [a2])
                    o2 = o2 + jnp.where(a1 == a2, 1, 0).astype(jnp.int32)
                    plsc.store_scatter(hists[u], [a1], o1 + ones)
                    plsc.store_scatter(hists[u], [a2], o2 + ones)
                    lrank[pl.ds((t * NHIST + u) * LANES, LANES)] = o1
                    lrank[pl.ds(((t + 1) * NHIST + u) * LANES, LANES)] = o2
                return carry
            lax.fori_loop(0, sub // 2, count, 0)

            @plsc.parallel_loop(0, RADIX, unroll=4)
            def s_tot(g):
                v = ((h0[pl.ds(g * LANES, LANES)] + h1[pl.ds(g * LANES, LANES)])
                     + (h2[pl.ds(g * LANES, LANES)] + h3[pl.ds(g * LANES, LANES)]))
                incl = plsc.cumsum(v)
                plsc.store_scatter(r_tot, [zeros16 + g], incl,
                                   mask=lane == LANES - 1)

            def s_scan(b, acc):
                v = r_tot[pl.ds(b * LANES, LANES)]
                incl = plsc.cumsum(v)
                r_exc[pl.ds(b * LANES, LANES)] = acc + (incl - v)
                return acc + jnp.max(incl)
            lax.fori_loop(0, RADIX // LANES, s_scan, zeros16, unroll=2)

            @plsc.parallel_loop(0, RADIX, unroll=4)
            def s_wb(g):
                base = plsc.load_gather(r_exc, [zeros16 + g])
                c0 = h0[pl.ds(g * LANES, LANES)]
                c1 = h1[pl.ds(g * LANES, LANES)]
                c2 = h2[pl.ds(g * LANES, LANES)]
                c3 = h3[pl.ds(g * LANES, LANES)]
                v = (c0 + c1) + (c2 + c3)
                exl = plsc.cumsum(v) - v
                pre0 = base + exl
                pre1 = pre0 + c0
                pre2 = pre1 + c1
                h0[pl.ds(g * LANES, LANES)] = pre0
                h1[pl.ds(g * LANES, LANES)] = pre1
                h2[pl.ds(g * LANES, LANES)] = pre2
                h3[pl.ds(g * LANES, LANES)] = pre2 + c2

            @plsc.parallel_loop(0, sub, unroll=4)
            def perm(t):
                for u in range(NHIST):
                    off = u * sub + t
                    kv = plsc.load_gather(src_key, [seg0p + off])
                    pv = seg0n + off if first else plsc.load_gather(src_pay, [seg0p + off])
                    dig = lax.shift_right_logical(kv, shift) & 0xFF
                    pre = plsc.load_gather(hists[u], [dig * LANES + lane])
                    lr = lrank[pl.ds((t * NHIST + u) * LANES, LANES)]
                    dest = pre + lr
                    if last:
                        plsc.store_scatter(dst_pay, [dest], pv)
                    else:
                        dest_p = dest + lax.shift_right_logical(dest, 9)
                        plsc.store_scatter(dst_key, [dest_p], kv)
                        plsc.store_scatter(dst_pay, [dest_p], pv)

        def row_body(rr, carry):
            row = base_row + rr
            pltpu.sync_copy(d_hbm.at[row], d_buf)

            def repack(t, mm):
                mn, mx = mm
                kv = plsc.bitcast(d_buf[pl.ds(t * LANES, LANES)], jnp.int32)
                key0[pl.ds(t * LANES + lax.shift_right_logical(t, 5), LANES)] = kv
                return jnp.minimum(mn, kv), jnp.maximum(mx, kv)
            mn, mx = lax.fori_loop(0, nvec, repack,
                                   (jnp.full((LANES,), 0x7FFFFFFF, jnp.int32),
                                    zeros16), unroll=2)
            minb = jnp.min(mn)
            maxb = jnp.max(mx)
            need4 = lax.shift_right_logical(minb, 24) != lax.shift_right_logical(maxb, 24)

            do_pass(key0, None, kb, pb, 0, True, False)
            do_pass(kb, pb, ka, pa, 8, False, False)
            do_pass(ka, pa, kb, pb, 16, False, False)

            @pl.when(rr > 0)
            def _():
                pltpu.make_async_copy(pay_fin, isort_hbm.at[row - 1], i_sem).wait()
                pltpu.make_async_copy(k_buf, k_hbm.at[row - 1], k_sem).wait()
                pltpu.make_async_copy(z_buf, z_hbm.at[row - 1], z_sem).wait()

            @pl.when(need4)
            def _():
                do_pass(kb, pb, None, pay_fin, 24, False, True)

            @pl.when(jnp.logical_not(need4))
            def _():
                @plsc.parallel_loop(0, nvec, unroll=2)
                def depad(t):
                    pay_fin[pl.ds(t * LANES, LANES)] = \
                        pb[pl.ds(t * LANES + lax.shift_right_logical(t, 5), LANES)]

            pltpu.async_copy(pay_fin, isort_hbm.at[row], i_sem)

            @plsc.parallel_loop(0, nvec, unroll=2)
            def inv(t):
                u = pay_fin[pl.ds(t * LANES, LANES)]
                plsc.store_scatter(k_buf, [u], t * LANES + lane)

            @plsc.parallel_loop(0, nvec, unroll=2)
            def zfill(t):
                z_buf[pl.ds(t * LANES, LANES)] = fzeros

            top16 = pay_fin[pl.ds(0, LANES)]
            plsc.store_scatter(z_buf, [top16], zvals)

            pltpu.async_copy(k_buf, k_hbm.at[row], k_sem)
            pltpu.async_copy(z_buf, z_hbm.at[row], z_sem)

            plsc.store_scatter(nearest, [zeros16 + rr], top16, mask=lane0)
            return carry
        lax.fori_loop(0, rows_w, row_body, 0)

        last_row = base_row + rows_w - 1
        pltpu.make_async_copy(pay_fin, isort_hbm.at[last_row], i_sem).wait()
        pltpu.make_async_copy(k_buf, k_hbm.at[last_row], k_sem).wait()
        pltpu.make_async_copy(z_buf, z_hbm.at[last_row], z_sem).wait()

        def xg(t, carry):
            pltpu.async_copy(
                c_hbm.at[nearest.at[pl.ds(t * XC, XC)]], xc_buf, sem).wait()
            pltpu.sync_copy(xc_buf, xc_hbm.at[pl.ds(base_row + t * XC, XC)])
            return carry
        lax.fori_loop(0, rows_w // XC, xg, 0)

    return pl.kernel(
        body,
        out_type=[
            jax.ShapeDtypeStruct((n_tok, n_unit), jnp.int32),
            jax.ShapeDtypeStruct((n_tok, n_unit), jnp.int32),
            jax.ShapeDtypeStruct((n_tok, n_unit), jnp.float32),
            jax.ShapeDtypeStruct((n_tok, n_feat), jnp.float32),
        ],
        mesh=mesh,
        scratch_types=[
            pltpu.VMEM((n_unit,), jnp.float32),
            pltpu.VMEM((LANES * pseg,), jnp.int32),
            pltpu.VMEM((LANES * pseg,), jnp.int32),
            pltpu.VMEM((LANES * pseg,), jnp.int32),
            pltpu.VMEM((LANES * pseg,), jnp.int32),
            pltpu.VMEM((LANES * pseg,), jnp.int32),
            pltpu.VMEM((n_unit,), jnp.int32),
            pltpu.VMEM((histn,), jnp.int32),
            pltpu.VMEM((histn,), jnp.int32),
            pltpu.VMEM((histn,), jnp.int32),
            pltpu.VMEM((histn,), jnp.int32),
            pltpu.VMEM((n_unit,), jnp.int32),
            pltpu.VMEM((RADIX,), jnp.int32),
            pltpu.VMEM((RADIX,), jnp.int32),
            pltpu.VMEM((n_unit,), jnp.int32),
            pltpu.VMEM((n_unit,), jnp.float32),
            pltpu.VMEM((rows_w,), jnp.int32),
            pltpu.VMEM((XC, n_feat), jnp.float32),
            pltpu.SemaphoreType.DMA,
            pltpu.SemaphoreType.DMA,
            pltpu.SemaphoreType.DMA,
            pltpu.SemaphoreType.DMA,
        ],
        compiler_params=pltpu.CompilerParams(needs_layout_passes=False),
        interpret=interpret,
    )


def kernel(x, c):
    x2 = jnp.sum(x * x, axis=-1, keepdims=True)
    c2 = jnp.sum(c * c, axis=-1)[None, :]
    d = _dist(x, x2, c, c2)
    sc = _make_sc_sort(N_TOK, N_UNIT, N_FEAT)
    i_sort, k, z, x_c = sc(d, c)
    return (d, i_sort, k, z, x_c)


# speedup vs baseline: 1.6824x; 1.1614x over previous
import functools

import jax
import jax.numpy as jnp
from jax import lax
from jax.experimental import pallas as pl
from jax.experimental.pallas import tpu as pltpu
from jax.experimental.pallas import tpu_sc as plsc

N_TOK = 4096
N_FEAT = 256
N_UNIT = 8192
TOPK_N = 10


BM = 256
BN = 2048


def _dist_body(x_ref, x2_ref, c_ref, c2_ref, o_ref):
    g = jax.lax.dot_general(
        x_ref[...], c_ref[...], (((1,), (1,)), ((), ())),
        preferred_element_type=jnp.float32,
    )
    d2 = jnp.maximum(x2_ref[...] - 2.0 * g + c2_ref[...], 0.0)
    o_ref[...] = jnp.sqrt(d2)


def _dist(x, x2, c, c2):
    grid = (N_TOK // BM, N_UNIT // BN)
    return pl.pallas_call(
        _dist_body,
        grid=grid,
        in_specs=[
            pl.BlockSpec((BM, N_FEAT), lambda i, j: (i, 0)),
            pl.BlockSpec((BM, 1), lambda i, j: (i, 0)),
            pl.BlockSpec((BN, N_FEAT), lambda i, j: (j, 0)),
            pl.BlockSpec((1, BN), lambda i, j: (0, j)),
        ],
        out_specs=pl.BlockSpec((BM, BN), lambda i, j: (i, j)),
        out_shape=jax.ShapeDtypeStruct((N_TOK, N_UNIT), jnp.float32),
    )(x, x2, c, c2)


NC = 2
NS = 16
NW = NC * NS
LANES = 16
RADIX = 256
NHIST = 4
XC = 32


def _make_sc_sort(n_tok, n_unit, n_feat, interpret=False):
    seg = n_unit // LANES
    pseg = seg + 1
    sub = seg // NHIST
    histn = RADIX * LANES
    nvec = n_unit // LANES
    vblk = seg // LANES
    rows_w = n_tok // NW

    mesh = plsc.VectorSubcoreMesh(
        core_axis_name="c", subcore_axis_name="s",
        num_cores=NC, num_subcores=NS)

    def body(d_hbm, c_hbm, isort_hbm, k_hbm, z_hbm, xc_hbm,
             d_buf, key0, ka, pa, kb, pb, pay_fin,
             h0, h1, h2, h3, lrank, r_tot, r_exc,
             k_buf, z_buf, nearest, xc_buf, sem,
             i_sem, k_sem, z_sem):
        hists = (h0, h1, h2, h3)
        cid = lax.axis_index("c")
        sid = lax.axis_index("s")
        wid = sid * NC + cid
        base_row = wid * rows_w

        lane = jnp.arange(LANES, dtype=jnp.int32)
        seg0n = lane * seg
        seg0p = lane * pseg
        ones = jnp.ones((LANES,), jnp.int32)
        zeros16 = jnp.zeros((LANES,), jnp.int32)
        lane0 = lane == 0
        zvals = jnp.where(lane < TOPK_N,
                          1.0 / (lane.astype(jnp.float32) + 1.0),
                          jnp.float32(0.0))
        fzeros = jnp.zeros((LANES,), jnp.float32)

        def do_pass(src_key, src_pay, dst_key, dst_pay, shift, first, last):
            @plsc.parallel_loop(0, RADIX, unroll=2)
            def zh(i):
                h0[pl.ds(i * LANES, LANES)] = zeros16
                h1[pl.ds(i * LANES, LANES)] = zeros16
                h2[pl.ds(i * LANES, LANES)] = zeros16
                h3[pl.ds(i * LANES, LANES)] = zeros16

            W = 4

            def count(i, carry):
                mn, mx = carry
                t = i * W
                addrs = []
                for u in range(NHIST):
                    for w in range(W):
                        idx = seg0p + (u * sub + t + w)
                        kv = plsc.load_gather(src_key, [idx])
                        if first:
                            mn = jnp.minimum(mn, kv)
                            mx = jnp.maximum(mx, kv)
                        dig = lax.shift_right_logical(kv, shift) & 0xFF
                        addrs.append(dig * LANES + lane)
                for u in range(NHIST):
                    a = addrs[W * u:W * u + W]
                    o = [plsc.load_gather(hists[u], [aw]) for aw in a]
                    for w in range(1, W):
                        fix = jnp.where(a[0] == a[w], 1, 0)
                        for w2 in range(1, w):
                            fix = fix + jnp.where(a[w2] == a[w], 1, 0)
                        o[w] = o[w] + fix.astype(jnp.int32)
                    for w in range(W):
                        plsc.store_scatter(hists[u], [a[w]], o[w] + ones)
                        lrank[pl.ds(((t + w) * NHIST + u) * LANES, LANES)] = o[w]
                return mn, mx
            mnmx = lax.fori_loop(
                0, sub // W, count,
                (jnp.full((LANES,), 0x7FFFFFFF, jnp.int32), zeros16))

            @plsc.parallel_loop(0, RADIX, unroll=4)
            def s_tot(g):
                v = ((h0[pl.ds(g * LANES, LANES)] + h1[pl.ds(g * LANES, LANES)])
                     + (h2[pl.ds(g * LANES, LANES)] + h3[pl.ds(g * LANES, LANES)]))
                incl = plsc.cumsum(v)
                plsc.store_scatter(r_tot, [zeros16 + g], incl,
                                   mask=lane == LANES - 1)

            def s_scan(b, acc):
                v = r_tot[pl.ds(b * LANES, LANES)]
                incl = plsc.cumsum(v)
                r_exc[pl.ds(b * LANES, LANES)] = acc + (incl - v)
                return acc + jnp.max(incl)
            lax.fori_loop(0, RADIX // LANES, s_scan, zeros16, unroll=2)

            @plsc.parallel_loop(0, RADIX, unroll=4)
            def s_wb(g):
                base = plsc.load_gather(r_exc, [zeros16 + g])
                c0 = h0[pl.ds(g * LANES, LANES)]
                c1 = h1[pl.ds(g * LANES, LANES)]
                c2 = h2[pl.ds(g * LANES, LANES)]
                c3 = h3[pl.ds(g * LANES, LANES)]
                v = (c0 + c1) + (c2 + c3)
                exl = plsc.cumsum(v) - v
                pre0 = base + exl
                pre1 = pre0 + c0
                pre2 = pre1 + c1
                h0[pl.ds(g * LANES, LANES)] = pre0
                h1[pl.ds(g * LANES, LANES)] = pre1
                h2[pl.ds(g * LANES, LANES)] = pre2
                h3[pl.ds(g * LANES, LANES)] = pre2 + c2

            @plsc.parallel_loop(0, sub, unroll=4)
            def perm(t):
                for u in range(NHIST):
                    off = u * sub + t
                    kv = plsc.load_gather(src_key, [seg0p + off])
                    pv = seg0n + off if first else plsc.load_gather(src_pay, [seg0p + off])
                    dig = lax.shift_right_logical(kv, shift) & 0xFF
                    pre = plsc.load_gather(hists[u], [dig * LANES + lane])
                    lr = lrank[pl.ds((t * NHIST + u) * LANES, LANES)]
                    dest = pre + lr
                    if last:
                        plsc.store_scatter(dst_pay, [dest], pv)
                    else:
                        dest_p = dest + lax.shift_right_logical(dest, 9)
                        plsc.store_scatter(dst_key, [dest_p], kv)
                        plsc.store_scatter(dst_pay, [dest_p], pv)
            return mnmx

        def row_body(rr, carry):
            row = base_row + rr
            pltpu.sync_copy(d_hbm.at[row], d_buf)

            @plsc.parallel_loop(0, nvec, unroll=4)
            def repack(t):
                kv = plsc.bitcast(d_buf[pl.ds(t * LANES, LANES)], jnp.int32)
                key0[pl.ds(t * LANES + lax.shift_right_logical(t, 5), LANES)] = kv

            mn, mx = do_pass(key0, None, kb, pb, 0, True, False)
            minb = jnp.min(mn)
            maxb = jnp.max(mx)
            need4 = lax.shift_right_logical(minb, 24) != lax.shift_right_logical(maxb, 24)

            do_pass(kb, pb, ka, pa, 8, False, False)
            do_pass(ka, pa, kb, pb, 16, False, False)

            @pl.when(rr > 0)
            def _():
                pltpu.make_async_copy(pay_fin, isort_hbm.at[row - 1], i_sem).wait()
                pltpu.make_async_copy(k_buf, k_hbm.at[row - 1], k_sem).wait()
                pltpu.make_async_copy(z_buf, z_hbm.at[row - 1], z_sem).wait()

            @pl.when(need4)
            def _():
                do_pass(kb, pb, None, pay_fin, 24, False, True)

            @pl.when(jnp.logical_not(need4))
            def _():
                @plsc.parallel_loop(0, nvec, unroll=2)
                def depad(t):
                    pay_fin[pl.ds(t * LANES, LANES)] = \
                        pb[pl.ds(t * LANES + lax.shift_right_logical(t, 5), LANES)]

            pltpu.async_copy(pay_fin, isort_hbm.at[row], i_sem)

            @plsc.parallel_loop(0, nvec, unroll=2)
            def inv(t):
                u = pay_fin[pl.ds(t * LANES, LANES)]
                plsc.store_scatter(k_buf, [u], t * LANES + lane)

            @plsc.parallel_loop(0, nvec, unroll=2)
            def zfill(t):
                z_buf[pl.ds(t * LANES, LANES)] = fzeros

            top16 = pay_fin[pl.ds(0, LANES)]
            plsc.store_scatter(z_buf, [top16], zvals)

            pltpu.async_copy(k_buf, k_hbm.at[row], k_sem)
            pltpu.async_copy(z_buf, z_hbm.at[row], z_sem)

            plsc.store_scatter(nearest, [zeros16 + rr], top16, mask=lane0)
            return carry
        lax.fori_loop(0, rows_w, row_body, 0)

        last_row = base_row + rows_w - 1
        pltpu.make_async_copy(pay_fin, isort_hbm.at[last_row], i_sem).wait()
        pltpu.make_async_copy(k_buf, k_hbm.at[last_row], k_sem).wait()
        pltpu.make_async_copy(z_buf, z_hbm.at[last_row], z_sem).wait()

        def xg(t, carry):
            pltpu.async_copy(
                c_hbm.at[nearest.at[pl.ds(t * XC, XC)]], xc_buf, sem).wait()
            pltpu.sync_copy(xc_buf, xc_hbm.at[pl.ds(base_row + t * XC, XC)])
            return carry
        lax.fori_loop(0, rows_w // XC, xg, 0)

    return pl.kernel(
        body,
        out_type=[
            jax.ShapeDtypeStruct((n_tok, n_unit), jnp.int32),
            jax.ShapeDtypeStruct((n_tok, n_unit), jnp.int32),
            jax.ShapeDtypeStruct((n_tok, n_unit), jnp.float32),
            jax.ShapeDtypeStruct((n_tok, n_feat), jnp.float32),
        ],
        mesh=mesh,
        scratch_types=[
            pltpu.VMEM((n_unit,), jnp.float32),
            pltpu.VMEM((LANES * pseg,), jnp.int32),
            pltpu.VMEM((LANES * pseg,), jnp.int32),
            pltpu.VMEM((LANES * pseg,), jnp.int32),
            pltpu.VMEM((LANES * pseg,), jnp.int32),
            pltpu.VMEM((LANES * pseg,), jnp.int32),
            pltpu.VMEM((n_unit,), jnp.int32),
            pltpu.VMEM((histn,), jnp.int32),
            pltpu.VMEM((histn,), jnp.int32),
            pltpu.VMEM((histn,), jnp.int32),
            pltpu.VMEM((histn,), jnp.int32),
            pltpu.VMEM((n_unit,), jnp.int32),
            pltpu.VMEM((RADIX,), jnp.int32),
            pltpu.VMEM((RADIX,), jnp.int32),
            pltpu.VMEM((n_unit,), jnp.int32),
            pltpu.VMEM((n_unit,), jnp.float32),
            pltpu.VMEM((rows_w,), jnp.int32),
            pltpu.VMEM((XC, n_feat), jnp.float32),
            pltpu.SemaphoreType.DMA,
            pltpu.SemaphoreType.DMA,
            pltpu.SemaphoreType.DMA,
            pltpu.SemaphoreType.DMA,
        ],
        compiler_params=pltpu.CompilerParams(needs_layout_passes=False),
        interpret=interpret,
    )


def kernel(x, c):
    x2 = jnp.sum(x * x, axis=-1, keepdims=True)
    c2 = jnp.sum(c * c, axis=-1)[None, :]
    d = _dist(x, x2, c, c2)
    sc = _make_sc_sort(N_TOK, N_UNIT, N_FEAT)
    i_sort, k, z, x_c = sc(d, c)
    return (d, i_sort, k, z, x_c)


# speedup vs baseline: 1.7662x; 1.0498x over previous
import functools

import jax
import jax.numpy as jnp
from jax import lax
from jax.experimental import pallas as pl
from jax.experimental.pallas import tpu as pltpu
from jax.experimental.pallas import tpu_sc as plsc

N_TOK = 4096
N_FEAT = 256
N_UNIT = 8192
TOPK_N = 10


BM = 256
BN = 2048


def _dist_body(x_ref, x2_ref, c_ref, c2_ref, o_ref):
    g = jax.lax.dot_general(
        x_ref[...], c_ref[...], (((1,), (1,)), ((), ())),
        preferred_element_type=jnp.float32,
    )
    d2 = jnp.maximum(x2_ref[...] - 2.0 * g + c2_ref[...], 0.0)
    o_ref[...] = jnp.sqrt(d2)


def _dist(x, x2, c, c2):
    grid = (N_TOK // BM, N_UNIT // BN)
    return pl.pallas_call(
        _dist_body,
        grid=grid,
        in_specs=[
            pl.BlockSpec((BM, N_FEAT), lambda i, j: (i, 0)),
            pl.BlockSpec((BM, 1), lambda i, j: (i, 0)),
            pl.BlockSpec((BN, N_FEAT), lambda i, j: (j, 0)),
            pl.BlockSpec((1, BN), lambda i, j: (0, j)),
        ],
        out_specs=pl.BlockSpec((BM, BN), lambda i, j: (i, j)),
        out_shape=jax.ShapeDtypeStruct((N_TOK, N_UNIT), jnp.float32),
    )(x, x2, c, c2)


NC = 2
NS = 16
NW = NC * NS
LANES = 16
RADIX = 256
NHIST = 4
XC = 32


def _make_sc_sort(n_tok, n_unit, n_feat, interpret=False):
    seg = n_unit // LANES
    pseg = seg + 1
    sub = seg // NHIST
    histn = RADIX * LANES
    nvec = n_unit // LANES
    vblk = seg // LANES
    rows_w = n_tok // NW

    mesh = plsc.VectorSubcoreMesh(
        core_axis_name="c", subcore_axis_name="s",
        num_cores=NC, num_subcores=NS)

    def body(d_hbm, c_hbm, isort_hbm, k_hbm, z_hbm, xc_hbm,
             d_buf, key0, ka, pa, kb, pb, pay_fin,
             h0, h1, h2, h3, lrank, r_tot, r_exc,
             k_buf, z_buf, nearest, xc_buf, sem,
             i_sem, k_sem, z_sem, in_sem):
        hists = (h0, h1, h2, h3)
        cid = lax.axis_index("c")
        sid = lax.axis_index("s")
        wid = sid * NC + cid
        base_row = wid * rows_w

        lane = jnp.arange(LANES, dtype=jnp.int32)
        seg0n = lane * seg
        seg0p = lane * pseg
        ones = jnp.ones((LANES,), jnp.int32)
        zeros16 = jnp.zeros((LANES,), jnp.int32)
        lane0 = lane == 0
        zvals = jnp.where(lane < TOPK_N,
                          1.0 / (lane.astype(jnp.float32) + 1.0),
                          jnp.float32(0.0))
        fzeros = jnp.zeros((LANES,), jnp.float32)

        def do_pass(src_key, src_pay, dst_key, dst_pay, shift, first, last):
            @plsc.parallel_loop(0, RADIX, unroll=2)
            def zh(i):
                h0[pl.ds(i * LANES, LANES)] = zeros16
                h1[pl.ds(i * LANES, LANES)] = zeros16
                h2[pl.ds(i * LANES, LANES)] = zeros16
                h3[pl.ds(i * LANES, LANES)] = zeros16

            W = 4

            def count(i, carry):
                mn, mx = carry
                t = i * W
                addrs = []
                for u in range(NHIST):
                    for w in range(W):
                        idx = seg0p + (u * sub + t + w)
                        kv = plsc.load_gather(src_key, [idx])
                        if first:
                            mn = jnp.minimum(mn, kv)
                            mx = jnp.maximum(mx, kv)
                        dig = lax.shift_right_logical(kv, shift) & 0xFF
                        addrs.append(dig * LANES + lane)
                for u in range(NHIST):
                    a = addrs[W * u:W * u + W]
                    o = [plsc.load_gather(hists[u], [aw]) for aw in a]
                    for w in range(1, W):
                        fix = jnp.where(a[0] == a[w], 1, 0)
                        for w2 in range(1, w):
                            fix = fix + jnp.where(a[w2] == a[w], 1, 0)
                        o[w] = o[w] + fix.astype(jnp.int32)
                    for w in range(W):
                        plsc.store_scatter(hists[u], [a[w]], o[w] + ones)
                        lrank[pl.ds(((t + w) * NHIST + u) * LANES, LANES)] = o[w]
                return mn, mx
            mnmx = lax.fori_loop(
                0, sub // W, count,
                (jnp.full((LANES,), 0x7FFFFFFF, jnp.int32), zeros16))

            @plsc.parallel_loop(0, RADIX, unroll=4)
            def s_tot(g):
                v = ((h0[pl.ds(g * LANES, LANES)] + h1[pl.ds(g * LANES, LANES)])
                     + (h2[pl.ds(g * LANES, LANES)] + h3[pl.ds(g * LANES, LANES)]))
                incl = plsc.cumsum(v)
                plsc.store_scatter(r_tot, [zeros16 + g], incl,
                                   mask=lane == LANES - 1)

            def s_scan(b, acc):
                v = r_tot[pl.ds(b * LANES, LANES)]
                incl = plsc.cumsum(v)
                r_exc[pl.ds(b * LANES, LANES)] = acc + (incl - v)
                return acc + jnp.max(incl)
            lax.fori_loop(0, RADIX // LANES, s_scan, zeros16, unroll=2)

            @plsc.parallel_loop(0, RADIX, unroll=4)
            def s_wb(g):
                base = plsc.load_gather(r_exc, [zeros16 + g])
                c0 = h0[pl.ds(g * LANES, LANES)]
                c1 = h1[pl.ds(g * LANES, LANES)]
                c2 = h2[pl.ds(g * LANES, LANES)]
                c3 = h3[pl.ds(g * LANES, LANES)]
                v = (c0 + c1) + (c2 + c3)
                exl = plsc.cumsum(v) - v
                pre0 = base + exl
                pre1 = pre0 + c0
                pre2 = pre1 + c1
                h0[pl.ds(g * LANES, LANES)] = pre0
                h1[pl.ds(g * LANES, LANES)] = pre1
                h2[pl.ds(g * LANES, LANES)] = pre2
                h3[pl.ds(g * LANES, LANES)] = pre2 + c2

            @plsc.parallel_loop(0, sub, unroll=4)
            def perm(t):
                for u in range(NHIST):
                    off = u * sub + t
                    kv = plsc.load_gather(src_key, [seg0p + off])
                    pv = seg0n + off if first else plsc.load_gather(src_pay, [seg0p + off])
                    dig = lax.shift_right_logical(kv, shift) & 0xFF
                    pre = plsc.load_gather(hists[u], [dig * LANES + lane])
                    lr = lrank[pl.ds((t * NHIST + u) * LANES, LANES)]
                    dest = pre + lr
                    if last:
                        plsc.store_scatter(dst_pay, [dest], pv)
                    else:
                        dest_p = dest + lax.shift_right_logical(dest, 9)
                        plsc.store_scatter(dst_key, [dest_p], kv)
                        plsc.store_scatter(dst_pay, [dest_p], pv)
            return mnmx

        def row_body(rr, carry):
            row = base_row + rr
            pltpu.make_async_copy(d_hbm.at[row], d_buf, in_sem).wait()

            @plsc.parallel_loop(0, nvec, unroll=4)
            def repack(t):
                kv = plsc.bitcast(d_buf[pl.ds(t * LANES, LANES)], jnp.int32)
                key0[pl.ds(t * LANES + lax.shift_right_logical(t, 5), LANES)] = kv

            @pl.when(rr + 1 < rows_w)
            def _():
                pltpu.async_copy(d_hbm.at[row + 1], d_buf, in_sem)

            mn, mx = do_pass(key0, None, kb, pb, 0, True, False)
            minb = jnp.min(mn)
            maxb = jnp.max(mx)
            need4 = lax.shift_right_logical(minb, 24) != lax.shift_right_logical(maxb, 24)

            do_pass(kb, pb, ka, pa, 8, False, False)
            do_pass(ka, pa, kb, pb, 16, False, False)

            @pl.when(rr > 0)
            def _():
                pltpu.make_async_copy(pay_fin, isort_hbm.at[row - 1], i_sem).wait()
                pltpu.make_async_copy(k_buf, k_hbm.at[row - 1], k_sem).wait()
                pltpu.make_async_copy(z_buf, z_hbm.at[row - 1], z_sem).wait()

            @pl.when(need4)
            def _():
                do_pass(kb, pb, None, pay_fin, 24, False, True)

            @pl.when(jnp.logical_not(need4))
            def _():
                @plsc.parallel_loop(0, nvec, unroll=2)
                def depad(t):
                    pay_fin[pl.ds(t * LANES, LANES)] = \
                        pb[pl.ds(t * LANES + lax.shift_right_logical(t, 5), LANES)]

            pltpu.async_copy(pay_fin, isort_hbm.at[row], i_sem)

            @plsc.parallel_loop(0, nvec, unroll=2)
            def inv(t):
                u = pay_fin[pl.ds(t * LANES, LANES)]
                plsc.store_scatter(k_buf, [u], t * LANES + lane)

            @plsc.parallel_loop(0, nvec, unroll=2)
            def zfill(t):
                z_buf[pl.ds(t * LANES, LANES)] = fzeros

            top16 = pay_fin[pl.ds(0, LANES)]
            plsc.store_scatter(z_buf, [top16], zvals)

            pltpu.async_copy(k_buf, k_hbm.at[row], k_sem)
            pltpu.async_copy(z_buf, z_hbm.at[row], z_sem)

            plsc.store_scatter(nearest, [zeros16 + rr], top16, mask=lane0)
            return carry
        pltpu.async_copy(d_hbm.at[base_row], d_buf, in_sem)
        lax.fori_loop(0, rows_w, row_body, 0)

        last_row = base_row + rows_w - 1
        pltpu.make_async_copy(pay_fin, isort_hbm.at[last_row], i_sem).wait()
        pltpu.make_async_copy(k_buf, k_hbm.at[last_row], k_sem).wait()
        pltpu.make_async_copy(z_buf, z_hbm.at[last_row], z_sem).wait()

        def xg(t, carry):
            pltpu.async_copy(
                c_hbm.at[nearest.at[pl.ds(t * XC, XC)]], xc_buf, sem).wait()
            pltpu.sync_copy(xc_buf, xc_hbm.at[pl.ds(base_row + t * XC, XC)])
            return carry
        lax.fori_loop(0, rows_w // XC, xg, 0)

    return pl.kernel(
        body,
        out_type=[
            jax.ShapeDtypeStruct((n_tok, n_unit), jnp.int32),
            jax.ShapeDtypeStruct((n_tok, n_unit), jnp.int32),
            jax.ShapeDtypeStruct((n_tok, n_unit), jnp.float32),
            jax.ShapeDtypeStruct((n_tok, n_feat), jnp.float32),
        ],
        mesh=mesh,
        scratch_types=[
            pltpu.VMEM((n_unit,), jnp.float32),
            pltpu.VMEM((LANES * pseg,), jnp.int32),
            pltpu.VMEM((LANES * pseg,), jnp.int32),
            pltpu.VMEM((LANES * pseg,), jnp.int32),
            pltpu.VMEM((LANES * pseg,), jnp.int32),
            pltpu.VMEM((LANES * pseg,), jnp.int32),
            pltpu.VMEM((n_unit,), jnp.int32),
            pltpu.VMEM((histn,), jnp.int32),
            pltpu.VMEM((histn,), jnp.int32),
            pltpu.VMEM((histn,), jnp.int32),
            pltpu.VMEM((histn,), jnp.int32),
            pltpu.VMEM((n_unit,), jnp.int32),
            pltpu.VMEM((RADIX,), jnp.int32),
            pltpu.VMEM((RADIX,), jnp.int32),
            pltpu.VMEM((n_unit,), jnp.int32),
            pltpu.VMEM((n_unit,), jnp.float32),
            pltpu.VMEM((rows_w,), jnp.int32),
            pltpu.VMEM((XC, n_feat), jnp.float32),
            pltpu.SemaphoreType.DMA,
            pltpu.SemaphoreType.DMA,
            pltpu.SemaphoreType.DMA,
            pltpu.SemaphoreType.DMA,
            pltpu.SemaphoreType.DMA,
        ],
        compiler_params=pltpu.CompilerParams(needs_layout_passes=False),
        interpret=interpret,
    )


def kernel(x, c):
    x2 = jnp.sum(x * x, axis=-1, keepdims=True)
    c2 = jnp.sum(c * c, axis=-1)[None, :]
    d = _dist(x, x2, c, c2)
    sc = _make_sc_sort(N_TOK, N_UNIT, N_FEAT)
    i_sort, k, z, x_c = sc(d, c)
    return (d, i_sort, k, z, x_c)
